# baseline (speedup 1.0000x reference)
import functools

import jax
import jax.numpy as jnp
from jax import lax
from jax.experimental import pallas as pl
from jax.experimental.pallas import tpu as pltpu
from jax.experimental.pallas import tpu_sc as plsc

D_MODEL = 64
N_BITS_PAD = 32
T_ROWS = 512
CHUNK = 512


def _lut_body(x_ref, t_ref, o_ref):
    xrow = x_ref[0]
    iot = jax.lax.broadcasted_iota(jnp.int32, (N_BITS_PAD, 1), 0)
    bits_t = ((xrow >> iot) & 1).astype(jnp.float32)
    o_ref[...] = jax.lax.dot_general(
        bits_t,
        t_ref[...],
        (((0,), (0,)), ((), ())),
        preferred_element_type=jnp.float32,
    )


def _build_lut(table):
    u7 = jnp.arange(128, dtype=jnp.int32)
    u6 = jnp.arange(64, dtype=jnp.int32)
    x_lut = jnp.concatenate(
        [u7, u7 << 7, u6 << 14, jnp.zeros(192, jnp.int32)]
    ).reshape(1, T_ROWS)
    tpad = jnp.zeros((N_BITS_PAD, D_MODEL), table.dtype).at[: table.shape[0]].set(table)
    return pl.pallas_call(
        _lut_body,
        grid=(1,),
        in_specs=[
            pl.BlockSpec((1, T_ROWS), lambda i: (0, 0)),
            pl.BlockSpec((N_BITS_PAD, D_MODEL), lambda i: (0, 0)),
        ],
        out_specs=pl.BlockSpec((T_ROWS, D_MODEL), lambda i: (0, 0)),
        out_shape=jax.ShapeDtypeStruct((T_ROWS, D_MODEL), jnp.float32),
    )(x_lut, tpad)


def _sc_kernel(n):
    info = plsc.get_sparse_core_info()
    nw = info.num_cores * info.num_subcores
    per_w = n // nw
    n_chunks = per_w // CHUNK
    mesh = plsc.VectorSubcoreMesh(core_axis_name="c", subcore_axis_name="s")

    @functools.partial(
        pl.kernel,
        mesh=mesh,
        out_type=jax.ShapeDtypeStruct((n * D_MODEL,), jnp.float32),
        scratch_types=[
            pltpu.VMEM((T_ROWS * D_MODEL,), jnp.float32),
            pltpu.VMEM((CHUNK,), jnp.int32),
            pltpu.VMEM((CHUNK * D_MODEL,), jnp.float32),
        ],
    )
    def k(x_hbm, t_hbm, out_hbm, t_v, x_v, out_v):
        wid = lax.axis_index("s") * info.num_cores + lax.axis_index("c")
        w_base = wid * per_w
        pltpu.sync_copy(t_hbm, t_v)

        def chunk_body(c, _):
            base = w_base + c * CHUNK
            pltpu.sync_copy(x_hbm.at[pl.ds(base, CHUNK)], x_v)

            def pos_body(g, _2):
                v = x_v[pl.ds(g * 16, 16)]
                for i in range(16):
                    s = v[i]
                    r = g * 16 + i
                    a0 = (s & 127) * D_MODEL
                    a1 = (128 + ((s >> 7) & 127)) * D_MODEL
                    a2 = (256 + ((s >> 14) & 63)) * D_MODEL
                    for j in range(D_MODEL // 16):
                        o = j * 16
                        out_v[pl.ds(r * D_MODEL + o, 16)] = (
                            t_v[pl.ds(a0 + o, 16)]
                            + t_v[pl.ds(a1 + o, 16)]
                            + t_v[pl.ds(a2 + o, 16)]
                        )
                return 0

            lax.fori_loop(0, CHUNK // 16, pos_body, 0)
            pltpu.sync_copy(out_v, out_hbm.at[pl.ds(base * D_MODEL, CHUNK * D_MODEL)])
            return 0

        lax.fori_loop(0, n_chunks, chunk_body, 0)

    return k


def kernel(x, table):
    x_shape = x.shape
    n = x.size
    xf = x.reshape(n)
    t = _build_lut(table).reshape(T_ROWS * D_MODEL)
    out = _sc_kernel(n)(xf, t)
    return out.reshape(*x_shape, D_MODEL)


# speedup vs baseline: 1.2167x; 1.2167x over previous
import functools

import jax
import jax.numpy as jnp
from jax import lax
from jax.experimental import pallas as pl
from jax.experimental.pallas import tpu as pltpu
from jax.experimental.pallas import tpu_sc as plsc

D_MODEL = 64
N_BITS_PAD = 32
T_ROWS = 512
CHUNK = 512


def _lut_body(x_ref, t_ref, o_ref):
    xrow = x_ref[0]
    iot = jax.lax.broadcasted_iota(jnp.int32, (N_BITS_PAD, 1), 0)
    bits_t = ((xrow >> iot) & 1).astype(jnp.float32)
    o_ref[...] = jax.lax.dot_general(
        bits_t,
        t_ref[...],
        (((0,), (0,)), ((), ())),
        preferred_element_type=jnp.float32,
    )


def _build_lut(table):
    u7 = jnp.arange(128, dtype=jnp.int32)
    u6 = jnp.arange(64, dtype=jnp.int32)
    x_lut = jnp.concatenate(
        [u7, u7 << 7, u6 << 14, jnp.zeros(192, jnp.int32)]
    ).reshape(1, T_ROWS)
    tpad = jnp.zeros((N_BITS_PAD, D_MODEL), table.dtype).at[: table.shape[0]].set(table)
    return pl.pallas_call(
        _lut_body,
        grid=(1,),
        in_specs=[
            pl.BlockSpec((1, T_ROWS), lambda i: (0, 0)),
            pl.BlockSpec((N_BITS_PAD, D_MODEL), lambda i: (0, 0)),
        ],
        out_specs=pl.BlockSpec((T_ROWS, D_MODEL), lambda i: (0, 0)),
        out_shape=jax.ShapeDtypeStruct((T_ROWS, D_MODEL), jnp.float32),
    )(x_lut, tpad)


def _sc_kernel(n):
    info = plsc.get_sparse_core_info()
    nw = info.num_cores * info.num_subcores
    per_w = n // nw
    n_chunks = per_w // CHUNK
    mesh = plsc.VectorSubcoreMesh(core_axis_name="c", subcore_axis_name="s")

    @functools.partial(
        pl.kernel,
        mesh=mesh,
        out_type=jax.ShapeDtypeStruct((n * D_MODEL,), jnp.float32),
        scratch_types=[
            pltpu.VMEM((T_ROWS * D_MODEL,), jnp.float32),
            pltpu.VMEM((CHUNK,), jnp.int32),
            pltpu.VMEM((CHUNK * D_MODEL,), jnp.float32),
        ],
    )
    def k(x_hbm, t_hbm, out_hbm, t_v, x_v, out_v):
        wid = lax.axis_index("s") * info.num_cores + lax.axis_index("c")
        w_base = wid * per_w
        pltpu.sync_copy(t_hbm, t_v)

        def chunk_body(c, _):
            base = w_base + c * CHUNK
            pltpu.sync_copy(x_hbm.at[pl.ds(base, CHUNK)], x_v)

            @plsc.parallel_loop(0, CHUNK // 16, 1, unroll=4)
            def pos_body(g):
                v = x_v[pl.ds(g * 16, 16)]
                for i in range(16):
                    s = v[i]
                    r = g * 16 + i
                    a0 = (s & 127) * D_MODEL
                    a1 = (128 + ((s >> 7) & 127)) * D_MODEL
                    a2 = (256 + ((s >> 14) & 63)) * D_MODEL
                    for j in range(D_MODEL // 16):
                        o = j * 16
                        out_v[pl.ds(r * D_MODEL + o, 16)] = (
                            t_v[pl.ds(a0 + o, 16)]
                            + t_v[pl.ds(a1 + o, 16)]
                            + t_v[pl.ds(a2 + o, 16)]
                        )
            pltpu.sync_copy(out_v, out_hbm.at[pl.ds(base * D_MODEL, CHUNK * D_MODEL)])
            return 0

        lax.fori_loop(0, n_chunks, chunk_body, 0)

    return k


def kernel(x, table):
    x_shape = x.shape
    n = x.size
    xf = x.reshape(n)
    t = _build_lut(table).reshape(T_ROWS * D_MODEL)
    out = _sc_kernel(n)(xf, t)
    return out.reshape(*x_shape, D_MODEL)
